# baseline (speedup 1.0000x reference)
import functools

import jax
import jax.numpy as jnp
from jax import lax
from jax.experimental import pallas as pl
from jax.experimental.pallas import tpu as pltpu
from jax.experimental.pallas import tpu_sc as plsc

_NB = 2
_CR = 16


def _make_sc_gather(V, D, B, S, NW, NB, CR):
    mesh = plsc.VectorSubcoreMesh(core_axis_name="c", subcore_axis_name="s")
    info = plsc.get_sparse_core_info()
    NC = info.num_cores
    rows_per_w = B // NW
    n_chunks = rows_per_w // CR

    @functools.partial(
        pl.kernel,
        mesh=mesh,
        compiler_params=pltpu.CompilerParams(use_tc_tiling_on_sc=False),
        out_type=jax.ShapeDtypeStruct((B, S, D), jnp.float32),
        scratch_types=[
            pltpu.VMEM((rows_per_w, S), jnp.int32),
            pltpu.VMEM((NB, CR, S, D), jnp.float32),
            pltpu.SemaphoreType.DMA((NB,)),
            pltpu.SemaphoreType.DMA((NB,)),
        ],
    )
    def gather(idx_hbm, table_hbm, out_hbm, idx_v, rows_v, gsem, ssem):
        wid = lax.axis_index("s") * NC + lax.axis_index("c")
        base = wid * rows_per_w
        pltpu.sync_copy(idx_hbm.at[pl.ds(base, rows_per_w)], idx_v)

        def g_start(b, j):
            for r in range(CR):
                pltpu.async_copy(
                    table_hbm.at[idx_v.at[j * CR + r]],
                    rows_v.at[b, r],
                    gsem.at[b],
                )

        def g_wait(b):
            pltpu.make_async_copy(
                out_hbm.at[pl.ds(0, CR)], rows_v.at[b], gsem.at[b]
            ).wait()

        def s_start(b, j):
            pltpu.async_copy(
                rows_v.at[b],
                out_hbm.at[pl.ds(base + j * CR, CR)],
                ssem.at[b],
            )

        def s_wait(b):
            pltpu.make_async_copy(
                rows_v.at[b], out_hbm.at[pl.ds(base, CR)], ssem.at[b]
            ).wait()

        for b in range(NB):
            g_start(b, b)
        for j in range(n_chunks):
            b = j % NB
            g_wait(b)
            s_start(b, j)
            if j + NB < n_chunks:
                s_wait(b)
                g_start(b, j + NB)
        for j in range(max(0, n_chunks - NB), n_chunks):
            s_wait(j % NB)

    return gather


def kernel(input, table):
    B, S = input.shape
    V, D = table.shape
    NW = 32
    idx = input if input.dtype == jnp.int32 else input.astype(jnp.int32)
    return _make_sc_gather(V, D, B, S, NW, _NB, _CR)(idx, table)
